# Initial kernel scaffold; baseline (speedup 1.0000x reference)
import jax
import jax.numpy as jnp
from jax.experimental import pallas as pl


def kernel(x_email, x_url, x_sender, ei_sender_email, ei_url_email, ei_email_url, We, be, Wu, bu, Ws, bs, Wl_se, bl_se, Wr_se, Wl_ue, bl_ue, Wr_ue, Wl_eu, bl_eu, Wr_eu, base_w, spline_w):
    raise NotImplementedError("write your pallas kernel here")


# speedup vs baseline: 7.8188x; 7.8188x over previous
import functools

import jax
import jax.numpy as jnp
from jax import lax
from jax.experimental import pallas as pl
from jax.experimental.pallas import tpu as pltpu
from jax.experimental.pallas import tpu_sc as plsc

N_EMAIL = 50000
H = 128
NROWS = 52000
ROWS_PER_SUB = NROWS // 16
CH = 128
NW = 32
NBUF = 8
UE_CHUNKS = 64
SE_CHUNKS = 28
UE_PAD = NW * UE_CHUNKS * CH
SE_PAD = NW * SE_CHUNKS * CH

def _seg_body(ue_src, ue_dst, se_src, se_dst, xurl, xs, zrows, out,
              iue_src, iue_dst, ise_src, ise_dst, rows,
              g0, g1, g2, g3, g4, g5, g6, g7,
              s0, s1, s2, s3, s4, s5, s6, s7, acc):
    gsems = [g0, g1, g2, g3, g4, g5, g6, g7]
    ssems = [s0, s1, s2, s3, s4, s5, s6, s7]
    c = lax.axis_index("c")
    s = lax.axis_index("s")
    wid = s * 2 + c

    pltpu.sync_copy(zrows, acc.at[pl.ds(s * ROWS_PER_SUB, ROWS_PER_SUB)])
    pltpu.sync_copy(ue_src.at[wid], iue_src)
    pltpu.sync_copy(ue_dst.at[wid], iue_dst)
    pltpu.sync_copy(se_src.at[wid], ise_src)
    pltpu.sync_copy(se_dst.at[wid], ise_dst)
    plsc.subcore_barrier()

    def _run(isrc, idst, table_hbm, nchunks, nbuf):
        for b in range(nbuf):
            pltpu.async_copy(table_hbm.at[isrc.at[b]], rows.at[b], gsems[b])

        def group(g, carry):
            jb = g * nbuf
            for b in range(nbuf):
                pltpu.make_async_copy(table_hbm.at[isrc.at[0]],
                                      rows.at[b], gsems[b]).wait()
                pltpu.async_copy(rows.at[b], acc.at[idst.at[jb + b]],
                                 ssems[b], add=True)
            for b in range(nbuf):
                @pl.when(jb + nbuf + b < nchunks)
                def _():
                    pltpu.make_async_copy(rows.at[b],
                                          acc.at[idst.at[0]], ssems[b]).wait()
                    pltpu.async_copy(table_hbm.at[isrc.at[jb + nbuf + b]],
                                     rows.at[b], gsems[b])
            return carry

        lax.fori_loop(0, nchunks // nbuf, group, 0)
        for b in range(nbuf):
            pltpu.make_async_copy(rows.at[b], acc.at[idst.at[0]],
                                  ssems[b]).wait()

    _run(iue_src, iue_dst, xurl, UE_CHUNKS, 8)
    _run(ise_src, ise_dst, xs, SE_CHUNKS, 4)

    plsc.subcore_barrier()
    pltpu.sync_copy(acc.at[pl.ds(s * ROWS_PER_SUB, ROWS_PER_SUB)],
                    out.at[c, pl.ds(s * ROWS_PER_SUB, ROWS_PER_SUB)])


@functools.lru_cache(maxsize=1)
def _seg_kernel():
    return functools.partial(
        pl.kernel,
        out_type=jax.ShapeDtypeStruct((2, NROWS, 16), jnp.float32),
        mesh=plsc.VectorSubcoreMesh(core_axis_name="c", subcore_axis_name="s",
                                    num_cores=2, num_subcores=16),
        compiler_params=pltpu.CompilerParams(use_tc_tiling_on_sc=False),
        scratch_types=(
            [
                pltpu.VMEM((UE_CHUNKS, CH), jnp.int32),
                pltpu.VMEM((UE_CHUNKS, CH), jnp.int32),
                pltpu.VMEM((SE_CHUNKS, CH), jnp.int32),
                pltpu.VMEM((SE_CHUNKS, CH), jnp.int32),
                pltpu.VMEM((NBUF, CH, 16), jnp.float32),
            ]
            + [pltpu.SemaphoreType.DMA] * (2 * NBUF)
            + [pltpu.VMEM_SHARED((NROWS, 16), jnp.float32)]
        ),
    )(_seg_body)


BLK = 2000


def _tc_body(xe_ref, pp_ref, wet_ref, be_ref, wr_ref, bl_ref, m_ref,
             bwt_ref, swt_ref, out_ref):
    he = jnp.dot(xe_ref[...], wet_ref[...],
                 preferred_element_type=jnp.float32) + be_ref[...]

    p = pp_ref[0, 0] + pp_ref[1, 0]
    col = lax.broadcasted_iota(jnp.int32, (BLK // 8, 16), 1)
    ws = []
    for q in range(8):
        sq = p[:, 16 * q:16 * q + 16]
        dq_ue = jnp.maximum(sq[:, 8:9], 1.0)
        dq_se = jnp.maximum(sq[:, 10:11], 1.0)
        denom = jnp.where(col <= 8, dq_ue, jnp.where(col <= 10, dq_se, 1.0))
        ws.append(sq / denom)
    w = jnp.stack(ws, axis=1).reshape(BLK, 16)

    m_e = jnp.dot(w, m_ref[...], preferred_element_type=jnp.float32)
    m_e = m_e + jnp.dot(he, wr_ref[...], preferred_element_type=jnp.float32)
    m_e = (m_e + bl_ref[...]) * 0.5
    h = jnp.where(m_e > 0, m_e, 0.2 * m_e) + he

    sig = 1.0 / (1.0 + jnp.exp(-h))
    acc = jnp.dot(h * sig, bwt_ref[...], preferred_element_type=jnp.float32)

    y = h * 2.5
    for k in range(8):
        m = y + (3.5 - k)
        rc = jnp.maximum(2.0 - jnp.abs(m), 0.0)
        p1 = (rc * rc) * rc * (1.0 / 6.0)
        p2 = ((-0.5 * rc + 2.0) * rc - 2.0) * rc + (2.0 / 3.0)
        bk = jnp.where(rc < 1.0, p1, p2)
        acc = acc + jnp.dot(bk, swt_ref[k], preferred_element_type=jnp.float32)

    out_ref[...] = acc


def _pad_edges(ei, total, chunks):
    pad = total - ei.shape[1]
    src = jnp.pad(ei[0], (0, pad))
    dst = jnp.pad(ei[1], (0, pad), constant_values=N_EMAIL)
    return src.reshape(NW, chunks, CH), dst.reshape(NW, chunks, CH)


def _prep_body(xu_ref, xs_ref, xu_out, xs_out):
    xu_out[:, 0:8] = xu_ref[...]
    xu_out[:, 8:9] = jnp.ones((2000, 1), jnp.float32)
    xu_out[:, 9:16] = jnp.zeros((2000, 7), jnp.float32)
    xs_out[:, 0:9] = jnp.zeros((400, 9), jnp.float32)
    xs_out[:, 9:10] = xs_ref[...]
    xs_out[:, 10:11] = jnp.ones((400, 1), jnp.float32)
    xs_out[:, 11:16] = jnp.zeros((400, 5), jnp.float32)


def _build_tables(x_url, x_sender):
    return pl.pallas_call(
        _prep_body,
        grid=(25,),
        in_specs=[
            pl.BlockSpec((2000, 8), lambda i: (i, 0)),
            pl.BlockSpec((400, 1), lambda i: (i, 0)),
        ],
        out_specs=[
            pl.BlockSpec((2000, 16), lambda i: (i, 0)),
            pl.BlockSpec((400, 16), lambda i: (i, 0)),
        ],
        out_shape=[
            jax.ShapeDtypeStruct((50000, 16), jnp.float32),
            jax.ShapeDtypeStruct((10000, 16), jnp.float32),
        ],
    )(x_url, x_sender)


def kernel(x_email, x_url, x_sender, ei_sender_email, ei_url_email,
           ei_email_url, We, be, Wu, bu, Ws, bs,
           Wl_se, bl_se, Wr_se, Wl_ue, bl_ue, Wr_ue, Wl_eu, bl_eu, Wr_eu,
           base_w, spline_w):
    ue_src, ue_dst = _pad_edges(ei_url_email, UE_PAD, UE_CHUNKS)
    se_src, se_dst = _pad_edges(ei_sender_email, SE_PAD, SE_CHUNKS)

    xurl_pad, xs_pad = _build_tables(x_url, x_sender)
    zrows = jnp.zeros((ROWS_PER_SUB, 16), jnp.float32)

    partials = _seg_kernel()(ue_src, ue_dst, se_src, se_dst,
                             xurl_pad, xs_pad, zrows)

    m_mat = jnp.zeros((16, H), jnp.float32)
    m_mat = m_mat.at[0:8].set((Wl_ue @ Wu).T)
    m_mat = m_mat.at[8].set(Wl_ue @ bu)
    m_mat = m_mat.at[9].set(Wl_se @ Ws[:, 0])
    m_mat = m_mat.at[10].set(Wl_se @ bs)

    wet = We.T
    wr = (Wr_se + Wr_ue).T
    bl2 = (bl_se + bl_ue).reshape(1, H)
    be2 = be.reshape(1, H)
    bwt = base_w.T
    swt = jnp.transpose(spline_w, (2, 1, 0))

    partials_packed = partials.reshape(2, NROWS * 16 // (BLK * 16), BLK // 8, 128)

    grid = N_EMAIL // BLK
    out = pl.pallas_call(
        _tc_body,
        grid=(grid,),
        in_specs=[
            pl.BlockSpec((BLK, 768), lambda i: (i, 0)),
            pl.BlockSpec((2, 1, BLK // 8, 128), lambda i: (0, i, 0, 0)),
            pl.BlockSpec((768, H), lambda i: (0, 0)),
            pl.BlockSpec((1, H), lambda i: (0, 0)),
            pl.BlockSpec((H, H), lambda i: (0, 0)),
            pl.BlockSpec((1, H), lambda i: (0, 0)),
            pl.BlockSpec((16, H), lambda i: (0, 0)),
            pl.BlockSpec((H, 2), lambda i: (0, 0)),
            pl.BlockSpec((8, H, 2), lambda i: (0, 0, 0)),
        ],
        out_specs=pl.BlockSpec((BLK, 2), lambda i: (i, 0)),
        out_shape=jax.ShapeDtypeStruct((N_EMAIL, 2), jnp.float32),
    )(x_email, partials_packed, wet, be2, wr, bl2, m_mat, bwt, swt)
    return out
